# baseline (speedup 1.0000x reference)
import jax
import jax.numpy as jnp
from jax.experimental import pallas as pl

N_E = 1024
E_DIM = 256
BETA = 0.25
TOK_TILE = 1024


def _vq_tile(zp_ref, ct_ref, csq_ref, zq_ref, idx_ref, part_ref):
    zp = zp_ref[...]
    ct = ct_ref[...]
    csq = csq_ref[...]
    m = jnp.dot(zp, ct)
    zsq = jnp.sum(zp * zp, axis=1, keepdims=True)
    d = (zsq + csq) - 2.0 * m
    mind = jnp.min(d, axis=1, keepdims=True)
    iota_k = jax.lax.broadcasted_iota(jnp.int32, d.shape, 1)
    big = jnp.int32(N_E)
    idx = jnp.min(jnp.where(d == mind, iota_k, big), axis=1, keepdims=True)
    onehot = jnp.where(iota_k == idx, 1.0, 0.0).astype(jnp.float32)
    zq_t = jax.lax.dot_general(ct, onehot, (((1,), (1,)), ((), ())))
    zq_ref[...] = zq_t[None]
    idx_ref[...] = idx[None]
    part_ref[...] = jnp.sum(mind).reshape(1, 1, 1)


def kernel(z, codebook):
    B, C, H, W = z.shape
    ntok = B * H * W
    ntile = ntok // TOK_TILE
    zp = jnp.transpose(z, (0, 2, 3, 1)).reshape(ntok, E_DIM)
    ct = codebook.T
    csq = jnp.sum(codebook ** 2, axis=1).reshape(1, N_E)

    grid = (ntile,)
    zq_t, idx, parts = pl.pallas_call(
        _vq_tile,
        grid=grid,
        in_specs=[
            pl.BlockSpec((TOK_TILE, E_DIM), lambda b: (b, 0)),
            pl.BlockSpec((E_DIM, N_E), lambda b: (0, 0)),
            pl.BlockSpec((1, N_E), lambda b: (0, 0)),
        ],
        out_specs=[
            pl.BlockSpec((1, E_DIM, TOK_TILE), lambda b: (b, 0, 0)),
            pl.BlockSpec((1, TOK_TILE, 1), lambda b: (b, 0, 0)),
            pl.BlockSpec((1, 1, 1), lambda b: (b, 0, 0)),
        ],
        out_shape=[
            jax.ShapeDtypeStruct((ntile, E_DIM, TOK_TILE), jnp.float32),
            jax.ShapeDtypeStruct((ntile, TOK_TILE, 1), jnp.int32),
            jax.ShapeDtypeStruct((ntile, 1, 1), jnp.float32),
        ],
    )(zp, ct, csq)

    z_q_out = zq_t.reshape(B, C, H, W)
    min_idx = idx.reshape(ntok)
    loss = (jnp.sum(parts) * ((1.0 + BETA) / float(ntok * E_DIM))).reshape(())
    return z_q_out, loss, min_idx
